# baseline (speedup 1.0000x reference)
import functools

import numpy as np
import jax
import jax.numpy as jnp
from jax import lax
from jax.experimental import pallas as pl
from jax.experimental.pallas import tpu as pltpu
from jax.experimental.pallas import tpu_sc as plsc

_LANES = 16
_TILES = 16
_CORES = 2
_CHUNK = 128


def _mm_scale(xp, w, d0, d1):
    npad, d = xp.shape
    u = w.shape[1]
    blk = npad // 4

    def body(x_ref, w_ref, d0_ref, d1_ref, h2_ref, dis_ref):
        h = jnp.dot(x_ref[...], w_ref[...], preferred_element_type=jnp.float32)
        dis = jax.lax.rsqrt(d0_ref[...] + d1_ref[...] + 1.0)
        h2_ref[...] = h * dis
        dis_ref[...] = dis

    return pl.pallas_call(
        body,
        grid=(4,),
        in_specs=[pl.BlockSpec((blk, d), lambda i: (i, 0)),
                  pl.BlockSpec((d, u), lambda i: (0, 0)),
                  pl.BlockSpec((blk, 1), lambda i: (i, 0)),
                  pl.BlockSpec((blk, 1), lambda i: (i, 0))],
        out_specs=[pl.BlockSpec((blk, u), lambda i: (i, 0)),
                   pl.BlockSpec((blk, 1), lambda i: (i, 0))],
        out_shape=[jax.ShapeDtypeStruct((npad, u), jnp.float32),
                   jax.ShapeDtypeStruct((npad, 1), jnp.float32)],
    )(xp, w, d0, d1)


def _final(accs, h2, dis, bias2d, n):
    _, npad, u = accs.shape
    blk = 2000

    def body(acc_ref, h2_ref, dis_ref, bias_ref, o_ref):
        acc = acc_ref[0] + acc_ref[1] - h2_ref[...]
        o_ref[...] = acc * dis_ref[...] + bias_ref[...]

    return pl.pallas_call(
        body,
        grid=(n // blk,),
        in_specs=[pl.BlockSpec((2, blk, u), lambda i: (0, i, 0)),
                  pl.BlockSpec((blk, u), lambda i: (i, 0)),
                  pl.BlockSpec((blk, 1), lambda i: (i, 0)),
                  pl.BlockSpec((1, u), lambda i: (0, 0))],
        out_specs=pl.BlockSpec((blk, u), lambda i: (i, 0)),
        out_shape=jax.ShapeDtypeStruct((n, u), jnp.float32),
    )(accs, h2, dis, bias2d)


def _hist(rowp, hn):
    ep = rowp.shape[0]
    nw = _CORES * _TILES
    hr = hn // 128
    per_tile = ep // nw
    steps = per_tile // _LANES
    mesh = plsc.VectorSubcoreMesh(core_axis_name="c", subcore_axis_name="s")

    @functools.partial(
        pl.kernel,
        out_type=jax.ShapeDtypeStruct((_CORES, hr, 128), jnp.float32),
        mesh=mesh,
        scratch_types=[
            pltpu.VMEM((per_tile,), jnp.int32),
            pltpu.VMEM((hr, 128), jnp.float32),
            pltpu.VMEM((hr,), jnp.int32),
            pltpu.VMEM_SHARED((hr, 128), jnp.float32),
        ],
        compiler_params=pltpu.CompilerParams(needs_layout_passes=False),
    )
    def hist_k(rowp_hbm, out_hbm, idxbuf, hist, iota_v, hacc):
        cid = lax.axis_index("c")
        sid = lax.axis_index("s")
        wid = cid * _TILES + sid
        pltpu.sync_copy(rowp_hbm.at[pl.ds(wid * per_tile, per_tile)], idxbuf)
        zeros = jnp.zeros((_LANES,), jnp.float32)
        ones = jnp.full((_LANES,), 1.0, jnp.float32)

        def zstep2(i, carry):
            for k in range(128 // _LANES):
                hist[i, pl.ds(k * _LANES, _LANES)] = zeros
            return carry

        lax.fori_loop(0, hr, zstep2, 0)
        for k in range(hr // _LANES):
            iota_v[pl.ds(k * _LANES, _LANES)] = (
                lax.iota(jnp.int32, _LANES) + k * _LANES)

        def step(i, carry):
            idx = idxbuf[pl.ds(i * _LANES, _LANES)]
            q = jax.lax.shift_right_logical(idx, 7)
            r = jax.lax.bitwise_and(idx, 127)
            plsc.addupdate_scatter(hist, [q, r], ones)
            return carry

        lax.fori_loop(0, steps, step, 0)
        @pl.when(sid == 0)
        def _():
            pltpu.sync_copy(hist, hacc)
        plsc.subcore_barrier()

        @pl.when(sid != 0)
        def _():
            pltpu.sync_copy(hist, hacc.at[iota_v], add=True)
        plsc.subcore_barrier()

        def writeout(slot):
            pltpu.sync_copy(hacc.at[pl.ds(sid * 8, 8)],
                            out_hbm.at[slot].at[pl.ds(sid * 8, 8)])

        @pl.when(sid < hr // 8)
        def _():
            @pl.when(cid == 0)
            def _():
                writeout(0)

            @pl.when(cid == 1)
            def _():
                writeout(1)

    return hist_k(rowp)


def _propagate(h2, rowp3, colp3):
    npad, u = h2.shape
    nw, nch, _ = rowp3.shape
    half = nch // 2
    rows_pt = npad // _TILES
    mesh = plsc.VectorSubcoreMesh(core_axis_name="c", subcore_axis_name="s")

    @functools.partial(
        pl.kernel,
        out_type=jax.ShapeDtypeStruct((_CORES, npad, u), jnp.float32),
        mesh=mesh,
        scratch_types=[
            pltpu.VMEM((half, _CHUNK), jnp.int32),
            pltpu.VMEM((half, _CHUNK), jnp.int32),
            pltpu.VMEM((_CHUNK, u), jnp.float32),
            pltpu.VMEM((_CHUNK, u), jnp.float32),
            pltpu.VMEM_SHARED((npad, u), jnp.float32),
            pltpu.SemaphoreType.DMA,
            pltpu.SemaphoreType.DMA,
        ],
        compiler_params=pltpu.CompilerParams(needs_layout_passes=False),
    )
    def prop_k(h2_hbm, rowp_hbm, colp_hbm, out_hbm,
               cbuf, rbuf, rows0, rows1, acc, sem0, sem1):
        cid = lax.axis_index("c")
        sid = lax.axis_index("s")

        def run(slot, out_slot):
            wid = slot * _TILES + sid
            pltpu.sync_copy(h2_hbm.at[pl.ds(sid * rows_pt, rows_pt)],
                            acc.at[pl.ds(sid * rows_pt, rows_pt)])
            plsc.subcore_barrier()

            for hh in range(2):
                pltpu.sync_copy(colp_hbm.at[wid].at[pl.ds(hh * half, half)], cbuf)
                pltpu.sync_copy(rowp_hbm.at[wid].at[pl.ds(hh * half, half)], rbuf)
                pltpu.async_copy(h2_hbm.at[cbuf.at[0]], rows0, sem0)

                def step(g, carry):
                    j0 = 2 * g
                    j1 = 2 * g + 1
                    j2 = lax.rem(2 * g + 2, half)
                    pltpu.make_async_copy(h2_hbm.at[cbuf.at[j0]], rows0, sem0).wait()
                    pltpu.async_copy(h2_hbm.at[cbuf.at[j1]], rows1, sem1)
                    pltpu.sync_copy(rows0, acc.at[rbuf.at[j0]], add=True)
                    pltpu.make_async_copy(h2_hbm.at[cbuf.at[j1]], rows1, sem1).wait()
                    pltpu.async_copy(h2_hbm.at[cbuf.at[j2]], rows0, sem0)
                    pltpu.sync_copy(rows1, acc.at[rbuf.at[j1]], add=True)
                    return carry

                lax.fori_loop(0, half // 2, step, 0)
                pltpu.make_async_copy(h2_hbm.at[cbuf.at[0]], rows0, sem0).wait()
            plsc.subcore_barrier()
            pltpu.sync_copy(acc.at[pl.ds(sid * rows_pt, rows_pt)],
                            out_hbm.at[out_slot].at[pl.ds(sid * rows_pt, rows_pt)])

        @pl.when(cid == 0)
        def _():
            run(0, 0)

        @pl.when(cid == 1)
        def _():
            run(1, 1)

    return prop_k(h2, rowp3, colp3)


def kernel(x, edge_index, kernel, bias):
    n, d = x.shape
    u = kernel.shape[1]
    e = edge_index.shape[1]

    group = _CORES * _TILES * _CHUNK * 4
    ep = ((e + group - 1) // group) * group
    nw = _CORES * _TILES
    nch = ep // (nw * _CHUNK)
    npad = ((n + 1 + 127) // 128) * 128
    hn = ((n + 1 + 1023) // 1024) * 1024

    row = edge_index[0]
    col = edge_index[1]
    xp = jnp.pad(x, ((0, npad - n), (0, 0)))

    per_tile = ep // nw
    ppt = per_tile - e // nw
    ar = np.arange(nw * ppt)
    prow = jnp.asarray((ar % n).reshape(nw, ppt).astype(np.int32))
    pcol = jnp.asarray((n + ar % (npad - n)).reshape(nw, ppt).astype(np.int32))
    rowp3 = jnp.concatenate([row.reshape(nw, e // nw), prow], axis=1)
    colp3 = jnp.concatenate([col.reshape(nw, e // nw), pcol], axis=1)
    rowp3 = rowp3.reshape(nw, nch, _CHUNK)
    colp3 = colp3.reshape(nw, nch, _CHUNK)

    eph = ((e + 511) // 512) * 512
    rowp_h = jnp.pad(row, (0, eph - e), constant_values=n)

    hist = _hist(rowp_h, hn)
    d0 = jnp.pad(hist[0].reshape(-1)[:n], (0, npad - n)).reshape(npad, 1)
    d1 = jnp.pad(hist[1].reshape(-1)[:n], (0, npad - n)).reshape(npad, 1)
    h2, dis = _mm_scale(xp, kernel, d0, d1)

    accs = _propagate(h2, rowp3, colp3)
    out = _final(accs, h2, dis, bias.reshape(1, u), n)
    return out


# speedup vs baseline: 1.0125x; 1.0125x over previous
import functools

import numpy as np
import jax
import jax.numpy as jnp
from jax import lax
from jax.experimental import pallas as pl
from jax.experimental.pallas import tpu as pltpu
from jax.experimental.pallas import tpu_sc as plsc

_LANES = 16
_TILES = 16
_CORES = 2
_CHUNK = 128


def _mm_scale(xp, w, d0, d1):
    npad, d = xp.shape
    u = w.shape[1]
    blk = npad // 4

    def body(x_ref, w_ref, d0_ref, d1_ref, h2_ref, dis_ref):
        h = jnp.dot(x_ref[...], w_ref[...], preferred_element_type=jnp.float32)
        dis = jax.lax.rsqrt(d0_ref[...] + d1_ref[...] + 1.0)
        h2_ref[...] = h * dis
        dis_ref[...] = dis

    return pl.pallas_call(
        body,
        grid=(4,),
        in_specs=[pl.BlockSpec((blk, d), lambda i: (i, 0)),
                  pl.BlockSpec((d, u), lambda i: (0, 0)),
                  pl.BlockSpec((blk, 1), lambda i: (i, 0)),
                  pl.BlockSpec((blk, 1), lambda i: (i, 0))],
        out_specs=[pl.BlockSpec((blk, u), lambda i: (i, 0)),
                   pl.BlockSpec((blk, 1), lambda i: (i, 0))],
        out_shape=[jax.ShapeDtypeStruct((npad, u), jnp.float32),
                   jax.ShapeDtypeStruct((npad, 1), jnp.float32)],
    )(xp, w, d0, d1)


def _final(accs, h2, dis, bias2d, n):
    _, npad, u = accs.shape
    blk = 2000

    def body(acc_ref, h2_ref, dis_ref, bias_ref, o_ref):
        acc = acc_ref[0] + acc_ref[1] - h2_ref[...]
        o_ref[...] = acc * dis_ref[...] + bias_ref[...]

    return pl.pallas_call(
        body,
        grid=(n // blk,),
        in_specs=[pl.BlockSpec((2, blk, u), lambda i: (0, i, 0)),
                  pl.BlockSpec((blk, u), lambda i: (i, 0)),
                  pl.BlockSpec((blk, 1), lambda i: (i, 0)),
                  pl.BlockSpec((1, u), lambda i: (0, 0))],
        out_specs=pl.BlockSpec((blk, u), lambda i: (i, 0)),
        out_shape=jax.ShapeDtypeStruct((n, u), jnp.float32),
    )(accs, h2, dis, bias2d)


def _hist(rowp, hn):
    ep = rowp.shape[0]
    nw = _CORES * _TILES
    hr = hn // 128
    per_tile = ep // nw
    steps = per_tile // _LANES
    mesh = plsc.VectorSubcoreMesh(core_axis_name="c", subcore_axis_name="s")

    @functools.partial(
        pl.kernel,
        out_type=jax.ShapeDtypeStruct((_CORES, hr, 128), jnp.float32),
        mesh=mesh,
        scratch_types=[
            pltpu.VMEM((per_tile,), jnp.int32),
            pltpu.VMEM((hr, 128), jnp.float32),
            pltpu.VMEM((hr,), jnp.int32),
            pltpu.VMEM_SHARED((hr, 128), jnp.float32),
        ],
        compiler_params=pltpu.CompilerParams(needs_layout_passes=False),
    )
    def hist_k(rowp_hbm, out_hbm, idxbuf, hist, iota_v, hacc):
        cid = lax.axis_index("c")
        sid = lax.axis_index("s")
        wid = cid * _TILES + sid
        pltpu.sync_copy(rowp_hbm.at[pl.ds(wid * per_tile, per_tile)], idxbuf)
        zeros = jnp.zeros((_LANES,), jnp.float32)
        ones = jnp.full((_LANES,), 1.0, jnp.float32)

        def zstep2(i, carry):
            for k in range(128 // _LANES):
                hist[i, pl.ds(k * _LANES, _LANES)] = zeros
            return carry

        lax.fori_loop(0, hr, zstep2, 0)
        for k in range(hr // _LANES):
            iota_v[pl.ds(k * _LANES, _LANES)] = (
                lax.iota(jnp.int32, _LANES) + k * _LANES)

        def step(i, carry):
            idx = idxbuf[pl.ds(i * _LANES, _LANES)]
            q = jax.lax.shift_right_logical(idx, 7)
            r = jax.lax.bitwise_and(idx, 127)
            plsc.addupdate_scatter(hist, [q, r], ones)
            return carry

        lax.fori_loop(0, steps, step, 0)
        @pl.when(sid == 0)
        def _():
            pltpu.sync_copy(hist, hacc)
        plsc.subcore_barrier()

        @pl.when(sid != 0)
        def _():
            pltpu.sync_copy(hist, hacc.at[iota_v], add=True)
        plsc.subcore_barrier()

        def writeout(slot):
            pltpu.sync_copy(hacc.at[pl.ds(sid * 8, 8)],
                            out_hbm.at[slot].at[pl.ds(sid * 8, 8)])

        @pl.when(sid < hr // 8)
        def _():
            @pl.when(cid == 0)
            def _():
                writeout(0)

            @pl.when(cid == 1)
            def _():
                writeout(1)

    return hist_k(rowp)


def _propagate(h2, main_r, main_c, rem_r, rem_c):
    npad, u = h2.shape
    nw, mc, _ = main_r.shape
    rc = rem_r.shape[1]
    half = mc // 2
    rows_pt = npad // _TILES
    mesh = plsc.VectorSubcoreMesh(core_axis_name="c", subcore_axis_name="s")

    @functools.partial(
        pl.kernel,
        out_type=jax.ShapeDtypeStruct((_CORES, npad, u), jnp.float32),
        mesh=mesh,
        scratch_types=[
            pltpu.VMEM((half, _CHUNK), jnp.int32),
            pltpu.VMEM((half, _CHUNK), jnp.int32),
            pltpu.VMEM((_CHUNK, u), jnp.float32),
            pltpu.VMEM((_CHUNK, u), jnp.float32),
            pltpu.VMEM_SHARED((npad, u), jnp.float32),
            pltpu.SemaphoreType.DMA,
            pltpu.SemaphoreType.DMA,
        ],
        compiler_params=pltpu.CompilerParams(needs_layout_passes=False),
    )
    def prop_k(h2_hbm, mr_hbm, mcol_hbm, rr_hbm, rcol_hbm, out_hbm,
               cbuf, rbuf, rows0, rows1, acc, sem0, sem1):
        cid = lax.axis_index("c")
        sid = lax.axis_index("s")

        def gwait(buf, sem):
            pltpu.make_async_copy(h2_hbm.at[cbuf.at[0]], buf, sem).wait()

        def pipe(nc):
            def step(g, carry):
                j1 = 2 * g + 1
                j2 = lax.rem(2 * g + 2, nc)
                gwait(rows0, sem0)
                pltpu.async_copy(h2_hbm.at[cbuf.at[j1]], rows1, sem1)
                pltpu.sync_copy(rows0, acc.at[rbuf.at[2 * g]], add=True)
                gwait(rows1, sem1)
                pltpu.async_copy(h2_hbm.at[cbuf.at[j2]], rows0, sem0)
                pltpu.sync_copy(rows1, acc.at[rbuf.at[j1]], add=True)
                return carry

            lax.fori_loop(0, nc // 2, step, 0)
            gwait(rows0, sem0)

        def run(slot, out_slot):
            wid = slot * _TILES + sid
            pltpu.sync_copy(rcol_hbm.at[wid], cbuf.at[pl.ds(0, rc)])
            pltpu.sync_copy(rr_hbm.at[wid], rbuf.at[pl.ds(0, rc)])
            pltpu.async_copy(h2_hbm.at[cbuf.at[0]], rows0, sem0)
            pltpu.sync_copy(h2_hbm.at[pl.ds(sid * rows_pt, rows_pt)],
                            acc.at[pl.ds(sid * rows_pt, rows_pt)])
            plsc.subcore_barrier()
            pipe(rc)

            for hh in range(2):
                pltpu.sync_copy(mcol_hbm.at[wid].at[pl.ds(hh * half, half)], cbuf)
                pltpu.sync_copy(mr_hbm.at[wid].at[pl.ds(hh * half, half)], rbuf)
                pltpu.async_copy(h2_hbm.at[cbuf.at[0]], rows0, sem0)
                pipe(half)
            plsc.subcore_barrier()
            pltpu.sync_copy(acc.at[pl.ds(sid * rows_pt, rows_pt)],
                            out_hbm.at[out_slot].at[pl.ds(sid * rows_pt, rows_pt)])

        @pl.when(cid == 0)
        def _():
            run(0, 0)

        @pl.when(cid == 1)
        def _():
            run(1, 1)

    return prop_k(h2, main_r, main_c, rem_r, rem_c)


def kernel(x, edge_index, kernel, bias):
    n, d = x.shape
    u = kernel.shape[1]
    e = edge_index.shape[1]

    group = _CORES * _TILES * _CHUNK * 4
    ep = ((e + group - 1) // group) * group
    nw = _CORES * _TILES
    nch = ep // (nw * _CHUNK)
    npad = ((n + 1 + 127) // 128) * 128
    hn = ((n + 1 + 1023) // 1024) * 1024

    row = edge_index[0]
    col = edge_index[1]
    xp = jnp.pad(x, ((0, npad - n), (0, 0)))

    mc = min((e // (nw * _CHUNK)) // 16 * 16, nch - 8)
    mcount = mc * nw * _CHUNK
    rc = nch - mc
    pad_cnt = rc * nw * _CHUNK - (e - mcount)
    ar = np.arange(pad_cnt)
    prow = jnp.asarray((ar % n).astype(np.int32))
    pcol = jnp.asarray((n + ar % (npad - n)).astype(np.int32))
    main_r = row[:mcount].reshape(nw, mc, _CHUNK)
    main_c = col[:mcount].reshape(nw, mc, _CHUNK)
    rem_r = jnp.concatenate([row[mcount:], prow]).reshape(nw, rc, _CHUNK)
    rem_c = jnp.concatenate([col[mcount:], pcol]).reshape(nw, rc, _CHUNK)

    eph = ((e + 511) // 512) * 512
    rowp_h = jnp.pad(row, (0, eph - e), constant_values=n)

    hist = _hist(rowp_h, hn)
    d0 = jnp.pad(hist[0].reshape(-1)[:n], (0, npad - n)).reshape(npad, 1)
    d1 = jnp.pad(hist[1].reshape(-1)[:n], (0, npad - n)).reshape(npad, 1)
    h2, dis = _mm_scale(xp, kernel, d0, d1)

    accs = _propagate(h2, main_r, main_c, rem_r, rem_c)
    out = _final(accs, h2, dis, bias.reshape(1, u), n)
    return out
